# baseline (speedup 1.0000x reference)
import functools

import jax
import jax.numpy as jnp
from jax import lax
from jax.experimental import pallas as pl
from jax.experimental.pallas import tpu as pltpu
from jax.experimental.pallas import tpu_sc as plsc

N = 10000
E = 320000
F = 128
H = 132

NC, NS = 2, 16
NW = NC * NS
CHUNK = 128
NBUF = 4
NCH = 80
NG = NCH // NBUF
EPT = NCH * CHUNK
EP = NW * EPT
ROWS = 10112
SLAB = ROWS // NS

RB = 400
GRID = N // RB


def _sc_degree():
    mesh = plsc.VectorSubcoreMesh(core_axis_name="c", subcore_axis_name="s")

    @functools.partial(
        pl.kernel,
        out_type=jax.ShapeDtypeStruct((NC, ROWS, F), jnp.float32),
        mesh=mesh,
        scratch_types=[
            pltpu.VMEM((NCH, CHUNK), jnp.int32),
            pltpu.VMEM((CHUNK, F), jnp.float32),
            pltpu.VMEM_SHARED((ROWS, F), jnp.float32),
        ],
    )
    def deg_kernel(dstp, zerosf, onesf, out, idx_d, ones_v, acc):
        c = lax.axis_index("c")
        s = lax.axis_index("s")
        wid = s * NC + c
        pltpu.sync_copy(zerosf.at[pl.ds(s * SLAB, SLAB)],
                        acc.at[pl.ds(s * SLAB, SLAB)])
        pltpu.sync_copy(dstp.at[wid], idx_d)
        pltpu.sync_copy(onesf, ones_v)
        plsc.subcore_barrier()

        def chunk(j, carry):
            pltpu.sync_copy(ones_v, acc.at[idx_d.at[j]], add=True)
            return carry

        lax.fori_loop(0, NCH, chunk, 0)
        plsc.subcore_barrier()
        pltpu.sync_copy(acc.at[pl.ds(s * SLAB, SLAB)],
                        out.at[c, pl.ds(s * SLAB, SLAB)])

    return deg_kernel


def _sc_scatter():
    mesh = plsc.VectorSubcoreMesh(core_axis_name="c", subcore_axis_name="s")

    @functools.partial(
        pl.kernel,
        out_type=jax.ShapeDtypeStruct((NC, ROWS, F), jnp.float32),
        mesh=mesh,
        scratch_types=[
            pltpu.VMEM((NCH, CHUNK), jnp.int32),
            pltpu.VMEM((NCH, CHUNK), jnp.int32),
            *[pltpu.VMEM((CHUNK, F), jnp.float32) for _ in range(NBUF)],
            pltpu.VMEM_SHARED((ROWS, F), jnp.float32),
        ],
    )
    def scat_kernel(y, srcp, dstp, zeros, out, idx_s, idx_d, b0, b1, b2, b3,
                    acc):
        buf = [b0, b1, b2, b3]
        c = lax.axis_index("c")
        s = lax.axis_index("s")
        wid = s * NC + c
        pltpu.sync_copy(zeros.at[pl.ds(s * SLAB, SLAB)],
                        acc.at[pl.ds(s * SLAB, SLAB)])
        pltpu.sync_copy(srcp.at[wid], idx_s)
        pltpu.sync_copy(dstp.at[wid], idx_d)
        plsc.subcore_barrier()

        def chunk(j, carry):
            pltpu.sync_copy(y.at[idx_s.at[j]], buf[0])
            pltpu.sync_copy(buf[0], acc.at[idx_d.at[j]], add=True)
            return carry

        lax.fori_loop(0, NCH, chunk, 0)
        plsc.subcore_barrier()
        pltpu.sync_copy(acc.at[pl.ds(s * SLAB, SLAB)],
                        out.at[c, pl.ds(s * SLAB, SLAB)])

    return scat_kernel


def _dinv_of(degp_ref):
    deg = degp_ref[0, :, 0] + degp_ref[1, :, 0] + 1.0
    return lax.rsqrt(deg)[:, None]


_DEG_SPEC = pl.BlockSpec((NC, RB, F), lambda i: (0, i, 0))
_ACC_SPEC = pl.BlockSpec((NC, RB, F), lambda i: (0, i, 0))
_ROW_SPEC = pl.BlockSpec((RB, F), lambda i: (i, 0))
_XP_SPEC = pl.BlockSpec((RB, 4), lambda i: (i, 0))


def _tc_pre(degp, x, wp):
    def body(d_ref, x_ref, wp_ref, y0_ref, xp_ref):
        xv = x_ref[...]
        y0_ref[...] = _dinv_of(d_ref) * xv
        xp_ref[...] = jnp.dot(xv, wp_ref[...],
                              preferred_element_type=jnp.float32)

    return pl.pallas_call(
        body,
        grid=(GRID,),
        in_specs=[_DEG_SPEC, _ROW_SPEC, pl.BlockSpec((F, 4), lambda i: (0, 0))],
        out_specs=[_ROW_SPEC, _XP_SPEC],
        out_shape=[jax.ShapeDtypeStruct((N, F), jnp.float32),
                   jax.ShapeDtypeStruct((N, 4), jnp.float32)],
    )(degp, x, wp)


def _tc_mid(degp, p1, y0, wcat, b1):
    def body(d_ref, p_ref, y0_ref, w_ref, b1_ref, g_ref, axp_ref):
        dinv = _dinv_of(d_ref)
        z1 = dinv * (p_ref[0] + p_ref[1] + y0_ref[...])
        zw = jnp.dot(z1, w_ref[...], preferred_element_type=jnp.float32)
        h = jnp.maximum(zw[:, :F] + b1_ref[...], 0.0)
        g_ref[...] = dinv * h
        axp_ref[...] = zw[:, F:F + 4]

    return pl.pallas_call(
        body,
        grid=(GRID,),
        in_specs=[
            _DEG_SPEC, _ACC_SPEC, _ROW_SPEC,
            pl.BlockSpec((F, F + 8), lambda i: (0, 0)),
            pl.BlockSpec((1, F), lambda i: (0, 0)),
        ],
        out_specs=[_ROW_SPEC, _XP_SPEC],
        out_shape=[jax.ShapeDtypeStruct((N, F), jnp.float32),
                   jax.ShapeDtypeStruct((N, 4), jnp.float32)],
    )(degp, p1, y0, wcat, b1)


def _tc_final(degp, q, g, axp, xproj, w2h, w2p, b2):
    def body(d_ref, q_ref, g_ref, axp_ref, xp_ref, w2h_ref, w2p_ref, b2_ref,
             o_ref):
        dinv = _dinv_of(d_ref)
        ah = dinv * (q_ref[0] + q_ref[1] + g_ref[...])
        out2 = jnp.dot(ah, w2h_ref[...], preferred_element_type=jnp.float32)
        out2 = out2 + jnp.dot(axp_ref[...], w2p_ref[...],
                              preferred_element_type=jnp.float32)
        out2 = out2 + b2_ref[...]
        o_ref[...] = jnp.concatenate([out2, xp_ref[...]], axis=1)

    return pl.pallas_call(
        body,
        grid=(GRID,),
        in_specs=[
            _DEG_SPEC, _ACC_SPEC, _ROW_SPEC, _XP_SPEC, _XP_SPEC,
            pl.BlockSpec((F, H), lambda i: (0, 0)),
            pl.BlockSpec((4, H), lambda i: (0, 0)),
            pl.BlockSpec((1, H), lambda i: (0, 0)),
        ],
        out_specs=pl.BlockSpec((RB, H + 4), lambda i: (i, 0)),
        out_shape=jax.ShapeDtypeStruct((N, H + 4), jnp.float32),
    )(degp, q, g, axp, xproj, w2h, w2p, b2)


def kernel(edge_index, x, Wp, W1, b1, W2, b2):
    src = edge_index[0]
    dst = edge_index[1]
    pad = EP - E
    srcp = jnp.concatenate(
        [src, jnp.zeros((pad,), jnp.int32)]).reshape(NW, NCH, CHUNK)
    dstp = jnp.concatenate(
        [dst, jnp.full((pad,), N, jnp.int32)]).reshape(NW, NCH, CHUNK)

    wcat = jnp.zeros((F, F + 8), jnp.float32)
    wcat = wcat.at[:, :F].set(W1).at[:, F:F + 4].set(Wp)
    b1r = b1.reshape(1, F)
    b2r = b2.reshape(1, H)
    w2h = W2[:F]
    w2p = W2[F:H]

    onesf = jnp.ones((CHUNK, F), jnp.float32)
    zerosf = jnp.zeros((ROWS, F), jnp.float32)

    degp = _sc_degree()(dstp, zerosf, onesf)
    y0, xproj = _tc_pre(degp, x, Wp)
    p1 = _sc_scatter()(y0, srcp, dstp, zerosf)
    g, axp = _tc_mid(degp, p1, y0, wcat, b1r)
    q = _sc_scatter()(g, srcp, dstp, zerosf)
    return _tc_final(degp, q, g, axp, xproj, w2h, w2p, b2r)


# speedup vs baseline: 1.0007x; 1.0007x over previous
import functools

import jax
import jax.numpy as jnp
from jax import lax
from jax.experimental import pallas as pl
from jax.experimental.pallas import tpu as pltpu
from jax.experimental.pallas import tpu_sc as plsc

N = 10000
E = 320000
F = 128
H = 132

NC, NS = 2, 16
NW = NC * NS
CHUNK = 128
NBUF = 4
NCH = 80
NG = NCH // NBUF
EPT = NCH * CHUNK
EP = NW * EPT
ROWS = 10112
SLAB = ROWS // NS

RB = 400
GRID = N // RB


def _sc_degree():
    mesh = plsc.VectorSubcoreMesh(core_axis_name="c", subcore_axis_name="s")

    @functools.partial(
        pl.kernel,
        out_type=jax.ShapeDtypeStruct((NC, ROWS, F), jnp.float32),
        mesh=mesh,
        scratch_types=[
            pltpu.VMEM((NCH, CHUNK), jnp.int32),
            pltpu.VMEM((CHUNK, F), jnp.float32),
            pltpu.VMEM_SHARED((ROWS, F), jnp.float32),
        ],
    )
    def deg_kernel(dstp, zerosf, onesf, out, idx_d, ones_v, acc):
        c = lax.axis_index("c")
        s = lax.axis_index("s")
        wid = s * NC + c
        pltpu.sync_copy(zerosf.at[pl.ds(s * SLAB, SLAB)],
                        acc.at[pl.ds(s * SLAB, SLAB)])
        pltpu.sync_copy(dstp.at[wid], idx_d)
        pltpu.sync_copy(onesf, ones_v)
        plsc.subcore_barrier()

        def chunk(j, carry):
            pltpu.sync_copy(ones_v, acc.at[idx_d.at[j]], add=True)
            return carry

        lax.fori_loop(0, NCH, chunk, 0)
        plsc.subcore_barrier()
        pltpu.sync_copy(acc.at[pl.ds(s * SLAB, SLAB)],
                        out.at[c, pl.ds(s * SLAB, SLAB)])

    return deg_kernel


def _sc_scatter():
    mesh = plsc.VectorSubcoreMesh(core_axis_name="c", subcore_axis_name="s")

    @functools.partial(
        pl.kernel,
        out_type=jax.ShapeDtypeStruct((NC, ROWS, F), jnp.float32),
        mesh=mesh,
        scratch_types=[
            pltpu.VMEM((NCH, CHUNK), jnp.int32),
            pltpu.VMEM((NCH, CHUNK), jnp.int32),
            *[pltpu.VMEM((CHUNK, F), jnp.float32) for _ in range(NBUF)],
            pltpu.SemaphoreType.DMA,
            pltpu.VMEM_SHARED((ROWS, F), jnp.float32),
        ],
    )
    def scat_kernel(y, srcp, dstp, zeros, out, idx_s, idx_d, b0, b1, b2, b3,
                    gsem, acc):
        buf = [b0, b1, b2, b3]
        c = lax.axis_index("c")
        s = lax.axis_index("s")
        wid = s * NC + c
        pltpu.sync_copy(zeros.at[pl.ds(s * SLAB, SLAB)],
                        acc.at[pl.ds(s * SLAB, SLAB)])
        pltpu.sync_copy(srcp.at[wid], idx_s)
        pltpu.sync_copy(dstp.at[wid], idx_d)
        plsc.subcore_barrier()

        def chunk(j, carry):
            pltpu.async_copy(y.at[idx_s.at[j]], buf[0], gsem).wait()
            pltpu.sync_copy(buf[0], acc.at[idx_d.at[j]], add=True)
            return carry

        lax.fori_loop(0, NCH, chunk, 0)
        plsc.subcore_barrier()
        pltpu.sync_copy(acc.at[pl.ds(s * SLAB, SLAB)],
                        out.at[c, pl.ds(s * SLAB, SLAB)])

    return scat_kernel


def _dinv_of(degp_ref):
    deg = degp_ref[0, :, 0] + degp_ref[1, :, 0] + 1.0
    return lax.rsqrt(deg)[:, None]


_DEG_SPEC = pl.BlockSpec((NC, RB, F), lambda i: (0, i, 0))
_ACC_SPEC = pl.BlockSpec((NC, RB, F), lambda i: (0, i, 0))
_ROW_SPEC = pl.BlockSpec((RB, F), lambda i: (i, 0))
_XP_SPEC = pl.BlockSpec((RB, 4), lambda i: (i, 0))


def _tc_pre(degp, x, wp):
    def body(d_ref, x_ref, wp_ref, y0_ref, xp_ref):
        xv = x_ref[...]
        y0_ref[...] = _dinv_of(d_ref) * xv
        xp_ref[...] = jnp.dot(xv, wp_ref[...],
                              preferred_element_type=jnp.float32)

    return pl.pallas_call(
        body,
        grid=(GRID,),
        in_specs=[_DEG_SPEC, _ROW_SPEC, pl.BlockSpec((F, 4), lambda i: (0, 0))],
        out_specs=[_ROW_SPEC, _XP_SPEC],
        out_shape=[jax.ShapeDtypeStruct((N, F), jnp.float32),
                   jax.ShapeDtypeStruct((N, 4), jnp.float32)],
    )(degp, x, wp)


def _tc_mid(degp, p1, y0, wcat, b1):
    def body(d_ref, p_ref, y0_ref, w_ref, b1_ref, g_ref, axp_ref):
        dinv = _dinv_of(d_ref)
        z1 = dinv * (p_ref[0] + p_ref[1] + y0_ref[...])
        zw = jnp.dot(z1, w_ref[...], preferred_element_type=jnp.float32)
        h = jnp.maximum(zw[:, :F] + b1_ref[...], 0.0)
        g_ref[...] = dinv * h
        axp_ref[...] = zw[:, F:F + 4]

    return pl.pallas_call(
        body,
        grid=(GRID,),
        in_specs=[
            _DEG_SPEC, _ACC_SPEC, _ROW_SPEC,
            pl.BlockSpec((F, F + 8), lambda i: (0, 0)),
            pl.BlockSpec((1, F), lambda i: (0, 0)),
        ],
        out_specs=[_ROW_SPEC, _XP_SPEC],
        out_shape=[jax.ShapeDtypeStruct((N, F), jnp.float32),
                   jax.ShapeDtypeStruct((N, 4), jnp.float32)],
    )(degp, p1, y0, wcat, b1)


def _tc_final(degp, q, g, axp, xproj, w2h, w2p, b2):
    def body(d_ref, q_ref, g_ref, axp_ref, xp_ref, w2h_ref, w2p_ref, b2_ref,
             o_ref):
        dinv = _dinv_of(d_ref)
        ah = dinv * (q_ref[0] + q_ref[1] + g_ref[...])
        out2 = jnp.dot(ah, w2h_ref[...], preferred_element_type=jnp.float32)
        out2 = out2 + jnp.dot(axp_ref[...], w2p_ref[...],
                              preferred_element_type=jnp.float32)
        out2 = out2 + b2_ref[...]
        o_ref[...] = jnp.concatenate([out2, xp_ref[...]], axis=1)

    return pl.pallas_call(
        body,
        grid=(GRID,),
        in_specs=[
            _DEG_SPEC, _ACC_SPEC, _ROW_SPEC, _XP_SPEC, _XP_SPEC,
            pl.BlockSpec((F, H), lambda i: (0, 0)),
            pl.BlockSpec((4, H), lambda i: (0, 0)),
            pl.BlockSpec((1, H), lambda i: (0, 0)),
        ],
        out_specs=pl.BlockSpec((RB, H + 4), lambda i: (i, 0)),
        out_shape=jax.ShapeDtypeStruct((N, H + 4), jnp.float32),
    )(degp, q, g, axp, xproj, w2h, w2p, b2)


def kernel(edge_index, x, Wp, W1, b1, W2, b2):
    src = edge_index[0]
    dst = edge_index[1]
    pad = EP - E
    srcp = jnp.concatenate(
        [src, jnp.zeros((pad,), jnp.int32)]).reshape(NW, NCH, CHUNK)
    dstp = jnp.concatenate(
        [dst, jnp.full((pad,), N, jnp.int32)]).reshape(NW, NCH, CHUNK)

    wcat = jnp.zeros((F, F + 8), jnp.float32)
    wcat = wcat.at[:, :F].set(W1).at[:, F:F + 4].set(Wp)
    b1r = b1.reshape(1, F)
    b2r = b2.reshape(1, H)
    w2h = W2[:F]
    w2p = W2[F:H]

    onesf = jnp.ones((CHUNK, F), jnp.float32)
    zerosf = jnp.zeros((ROWS, F), jnp.float32)

    degp = _sc_degree()(dstp, zerosf, onesf)
    y0, xproj = _tc_pre(degp, x, Wp)
    p1 = _sc_scatter()(y0, srcp, dstp, zerosf)
    g, axp = _tc_mid(degp, p1, y0, wcat, b1r)
    q = _sc_scatter()(g, srcp, dstp, zerosf)
    return _tc_final(degp, q, g, axp, xproj, w2h, w2p, b2r)


# speedup vs baseline: 2.1385x; 2.1370x over previous
import functools

import jax
import jax.numpy as jnp
from jax import lax
from jax.experimental import pallas as pl
from jax.experimental.pallas import tpu as pltpu
from jax.experimental.pallas import tpu_sc as plsc

N = 10000
E = 320000
F = 128
H = 132

NC, NS = 2, 16
NW = NC * NS
CHUNK = 128
NBUF = 4
NCH = 80
NG = NCH // NBUF
EPT = NCH * CHUNK
EP = NW * EPT
ROWS = 10112
SLAB = ROWS // NS

RB = 400
GRID = N // RB


def _sc_degree():
    mesh = plsc.VectorSubcoreMesh(core_axis_name="c", subcore_axis_name="s")

    @functools.partial(
        pl.kernel,
        out_type=jax.ShapeDtypeStruct((NC, ROWS, F), jnp.float32),
        mesh=mesh,
        scratch_types=[
            pltpu.VMEM((NCH, CHUNK), jnp.int32),
            pltpu.VMEM((CHUNK, F), jnp.float32),
            pltpu.VMEM_SHARED((ROWS, F), jnp.float32),
        ],
    )
    def deg_kernel(dstp, zerosf, onesf, out, idx_d, ones_v, acc):
        c = lax.axis_index("c")
        s = lax.axis_index("s")
        wid = s * NC + c
        pltpu.sync_copy(zerosf.at[pl.ds(s * SLAB, SLAB)],
                        acc.at[pl.ds(s * SLAB, SLAB)])
        pltpu.sync_copy(dstp.at[wid], idx_d)
        pltpu.sync_copy(onesf, ones_v)
        plsc.subcore_barrier()

        def chunk(j, carry):
            pltpu.sync_copy(ones_v, acc.at[idx_d.at[j]], add=True)
            return carry

        lax.fori_loop(0, NCH, chunk, 0)
        plsc.subcore_barrier()
        pltpu.sync_copy(acc.at[pl.ds(s * SLAB, SLAB)],
                        out.at[c, pl.ds(s * SLAB, SLAB)])

    return deg_kernel


def _sc_scatter():
    mesh = plsc.VectorSubcoreMesh(core_axis_name="c", subcore_axis_name="s")

    @functools.partial(
        pl.kernel,
        out_type=jax.ShapeDtypeStruct((NC, ROWS, F), jnp.float32),
        mesh=mesh,
        scratch_types=[
            pltpu.VMEM((NCH, CHUNK), jnp.int32),
            pltpu.VMEM((NCH, CHUNK), jnp.int32),
            *[pltpu.VMEM((CHUNK, F), jnp.float32) for _ in range(NBUF)],
            pltpu.SemaphoreType.DMA,
            pltpu.VMEM_SHARED((ROWS, F), jnp.float32),
        ],
    )
    def scat_kernel(y, srcp, dstp, zeros, out, idx_s, idx_d, b0, b1, b2, b3,
                    gsem, acc):
        buf = [b0, b1, b2, b3]
        c = lax.axis_index("c")
        s = lax.axis_index("s")
        wid = s * NC + c
        pltpu.sync_copy(zeros.at[pl.ds(s * SLAB, SLAB)],
                        acc.at[pl.ds(s * SLAB, SLAB)])
        pltpu.sync_copy(srcp.at[wid], idx_s)
        pltpu.sync_copy(dstp.at[wid], idx_d)
        plsc.subcore_barrier()

        def chunk(j, carry):
            pltpu.async_copy(y.at[idx_s.at[j]], buf[0], gsem).wait()
            pltpu.sync_copy(buf[0], acc.at[idx_d.at[j]], add=True)
            return carry

        lax.fori_loop(0, NCH, chunk, 0)
        plsc.subcore_barrier()
        pltpu.sync_copy(acc.at[pl.ds(s * SLAB, SLAB)],
                        out.at[c, pl.ds(s * SLAB, SLAB)])

    return scat_kernel


def _dinv_of(degp_ref):
    deg = degp_ref[0, :, 0] + degp_ref[1, :, 0] + 1.0
    return lax.rsqrt(deg)[:, None]


_DEG_SPEC = pl.BlockSpec((NC, RB, F), lambda i: (0, i, 0))
_ACC_SPEC = pl.BlockSpec((NC, RB, F), lambda i: (0, i, 0))
_ROW_SPEC = pl.BlockSpec((RB, F), lambda i: (i, 0))
_XP_SPEC = pl.BlockSpec((RB, 4), lambda i: (i, 0))


def _tc_pre(degp, x, wp):
    def body(d_ref, x_ref, wp_ref, y0_ref, xp_ref):
        xv = x_ref[...]
        y0_ref[...] = _dinv_of(d_ref) * xv
        xp_ref[...] = jnp.dot(xv, wp_ref[...],
                              preferred_element_type=jnp.float32)

    return pl.pallas_call(
        body,
        grid=(GRID,),
        in_specs=[_DEG_SPEC, _ROW_SPEC, pl.BlockSpec((F, 4), lambda i: (0, 0))],
        out_specs=[_ROW_SPEC, _XP_SPEC],
        out_shape=[jax.ShapeDtypeStruct((N, F), jnp.float32),
                   jax.ShapeDtypeStruct((N, 4), jnp.float32)],
    )(degp, x, wp)


def _tc_mid(degp, p1, y0, wcat, b1):
    def body(d_ref, p_ref, y0_ref, w_ref, b1_ref, g_ref, axp_ref):
        dinv = _dinv_of(d_ref)
        z1 = dinv * (p_ref[0] + p_ref[1] + y0_ref[...])
        zw = jnp.dot(z1, w_ref[...], preferred_element_type=jnp.float32)
        h = jnp.maximum(zw[:, :F] + b1_ref[...], 0.0)
        g_ref[...] = dinv * h
        axp_ref[...] = zw[:, F:F + 4]

    return pl.pallas_call(
        body,
        grid=(GRID,),
        in_specs=[
            _DEG_SPEC, _ACC_SPEC, _ROW_SPEC,
            pl.BlockSpec((F, F + 8), lambda i: (0, 0)),
            pl.BlockSpec((1, F), lambda i: (0, 0)),
        ],
        out_specs=[_ROW_SPEC, _XP_SPEC],
        out_shape=[jax.ShapeDtypeStruct((N, F), jnp.float32),
                   jax.ShapeDtypeStruct((N, 4), jnp.float32)],
    )(degp, p1, y0, wcat, b1)


def _tc_final(degp, q, g, axp, xproj, w2h, w2p, b2):
    def body(d_ref, q_ref, g_ref, axp_ref, xp_ref, w2h_ref, w2p_ref, b2_ref,
             o_ref):
        dinv = _dinv_of(d_ref)
        ah = dinv * (q_ref[0] + q_ref[1] + g_ref[...])
        out2 = jnp.dot(ah, w2h_ref[...], preferred_element_type=jnp.float32)
        out2 = out2 + jnp.dot(axp_ref[...], w2p_ref[...],
                              preferred_element_type=jnp.float32)
        out2 = out2 + b2_ref[...]
        o_ref[...] = jnp.concatenate([out2, xp_ref[...]], axis=1)

    return pl.pallas_call(
        body,
        grid=(GRID,),
        in_specs=[
            _DEG_SPEC, _ACC_SPEC, _ROW_SPEC, _XP_SPEC, _XP_SPEC,
            pl.BlockSpec((F, H), lambda i: (0, 0)),
            pl.BlockSpec((4, H), lambda i: (0, 0)),
            pl.BlockSpec((1, H), lambda i: (0, 0)),
        ],
        out_specs=pl.BlockSpec((RB, H + 4), lambda i: (i, 0)),
        out_shape=jax.ShapeDtypeStruct((N, H + 4), jnp.float32),
    )(degp, q, g, axp, xproj, w2h, w2p, b2)


def kernel(edge_index, x, Wp, W1, b1, W2, b2):
    src = edge_index[0]
    dst = edge_index[1]
    ppt = EPT - E // NW
    pad_dst = (N + jnp.arange(NW * ppt, dtype=jnp.int32) % (ROWS - N)
               ).reshape(NW, ppt)
    pad_src = (jnp.arange(NW * ppt, dtype=jnp.int32) % N).reshape(NW, ppt)
    srcp = jnp.concatenate(
        [src.reshape(NW, E // NW), pad_src], axis=1).reshape(NW, NCH, CHUNK)
    dstp = jnp.concatenate(
        [dst.reshape(NW, E // NW), pad_dst], axis=1).reshape(NW, NCH, CHUNK)

    wcat = jnp.zeros((F, F + 8), jnp.float32)
    wcat = wcat.at[:, :F].set(W1).at[:, F:F + 4].set(Wp)
    b1r = b1.reshape(1, F)
    b2r = b2.reshape(1, H)
    w2h = W2[:F]
    w2p = W2[F:H]

    onesf = jnp.ones((CHUNK, F), jnp.float32)
    zerosf = jnp.zeros((ROWS, F), jnp.float32)

    degp = _sc_degree()(dstp, zerosf, onesf)
    y0, xproj = _tc_pre(degp, x, Wp)
    p1 = _sc_scatter()(y0, srcp, dstp, zerosf)
    g, axp = _tc_mid(degp, p1, y0, wcat, b1r)
    q = _sc_scatter()(g, srcp, dstp, zerosf)
    return _tc_final(degp, q, g, axp, xproj, w2h, w2p, b2r)
